# Initial kernel scaffold; baseline (speedup 1.0000x reference)
import jax
import jax.numpy as jnp
from jax.experimental import pallas as pl


def kernel(x, emb0, emb1, emb2, emb3, emb4, emb5, emb6, emb7, emb8, emb9, emb10, fc1_W0, fc1_b0, fc1_g0, fc1_be0, fc1_W1, fc1_b1, fc1_g1, fc1_be1, fc1_W2, fc1_b2, fc1_g2, fc1_be2, fc1_W3, fc1_b3, fc2_W0, fc2_b0, fc2_g0, fc2_be0, fc2_W1, fc2_b1, fc2_g1, fc2_be1, fc2_W2, fc2_b2, fc2_g2, fc2_be2, fc2_W3, fc2_b3):
    raise NotImplementedError("write your pallas kernel here")


# speedup vs baseline: 3.7404x; 3.7404x over previous
import functools

import jax
import jax.numpy as jnp
from jax import lax
from jax.experimental import pallas as pl

B = 4096
EMB = 64
NT = 11
IDX = 24
K1_IN = NT * IDX
H = 1000
TB = 256
NBT = B // TB
EPS = 1e-5


def _prep_kernel(t_ref, w1_ref, w2_ref, p_ref):
    for t, w_ref in ((0, w1_ref), (1, w2_ref)):
        rows = []
        for j in range(NT):
            tj = t_ref[j * IDX:(j + 1) * IDX, :]
            wj = w_ref[j * EMB:(j + 1) * EMB, :]
            rows.append(jnp.dot(tj, wj, preferred_element_type=jnp.float32))
        p_ref[t] = jnp.concatenate(rows, axis=0)


def _l1_kernel(x_ref, p_ref, b_ref, raw_ref, st_ref):
    xt = x_ref[...]
    iota = lax.broadcasted_iota(jnp.int32, (TB, K1_IN), 1)
    oh = jnp.zeros((TB, K1_IN), jnp.float32)
    for j in range(NT):
        oh += (iota == xt[:, j:j + 1] + IDX * j).astype(jnp.float32)
    bi = pl.program_id(0)
    for t in range(2):
        raw = jnp.dot(oh, p_ref[t], preferred_element_type=jnp.float32)
        raw = raw + b_ref[t]
        raw_ref[t] = raw
        s = jnp.sum(raw, axis=0, keepdims=True)
        sq = jnp.sum(raw * raw, axis=0, keepdims=True)
        upd = jnp.concatenate([s, sq], axis=0)

        @pl.when(bi == 0)
        def _():
            st_ref[t] = upd

        @pl.when(bi != 0)
        def _():
            st_ref[t] = st_ref[t] + upd


def _bn_relu(raw, st, g, be):
    m = st[0:1] * (1.0 / B)
    ex2 = st[1:2] * (1.0 / B)
    v = ex2 - m * m
    scale = g * lax.rsqrt(v + EPS)
    shift = be - m * scale
    return jnp.maximum(raw * scale + shift, 0.0)


def _mid_kernel(raw_ref, st_ref, g_ref, be_ref, w_ref, b_ref,
                out_ref, ost_ref):
    hn = _bn_relu(raw_ref[0], st_ref[0], g_ref[0], be_ref[0])
    raw = jnp.dot(hn, w_ref[0], preferred_element_type=jnp.float32)
    raw = raw + b_ref[0]
    out_ref[0] = raw
    s = jnp.sum(raw, axis=0, keepdims=True)
    sq = jnp.sum(raw * raw, axis=0, keepdims=True)
    upd = jnp.concatenate([s, sq], axis=0)
    bi = pl.program_id(1)

    @pl.when(bi == 0)
    def _():
        ost_ref[0] = upd

    @pl.when(bi != 0)
    def _():
        ost_ref[0] = ost_ref[0] + upd


def _last_kernel(raw_ref, st_ref, g_ref, be_ref, w_ref, b_ref, out_ref):
    hn = _bn_relu(raw_ref[0], st_ref[0], g_ref[0], be_ref[0])
    out = jnp.dot(hn, w_ref[0], preferred_element_type=jnp.float32)
    out_ref[0] = out + b_ref[0]


def kernel(x, emb0, emb1, emb2, emb3, emb4, emb5, emb6, emb7, emb8, emb9,
           emb10,
           fc1_W0, fc1_b0, fc1_g0, fc1_be0,
           fc1_W1, fc1_b1, fc1_g1, fc1_be1,
           fc1_W2, fc1_b2, fc1_g2, fc1_be2,
           fc1_W3, fc1_b3,
           fc2_W0, fc2_b0, fc2_g0, fc2_be0,
           fc2_W1, fc2_b1, fc2_g1, fc2_be1,
           fc2_W2, fc2_b2, fc2_g2, fc2_be2,
           fc2_W3, fc2_b3):
    embs = [emb0, emb1, emb2, emb3, emb4, emb5, emb6, emb7, emb8, emb9,
            emb10]
    t_all = jnp.concatenate([e[:IDX] for e in embs], axis=0)

    f32 = jnp.float32

    p = pl.pallas_call(
        _prep_kernel,
        out_shape=jax.ShapeDtypeStruct((2, K1_IN, H), f32),
        in_specs=[
            pl.BlockSpec((K1_IN, EMB), lambda: (0, 0)),
            pl.BlockSpec((NT * EMB, H), lambda: (0, 0)),
            pl.BlockSpec((NT * EMB, H), lambda: (0, 0)),
        ],
        out_specs=pl.BlockSpec((2, K1_IN, H), lambda: (0, 0, 0)),
    )(t_all, fc1_W0, fc2_W0)

    b0 = jnp.stack([fc1_b0, fc2_b0]).reshape(2, 1, H)

    raw1, st1 = pl.pallas_call(
        _l1_kernel,
        grid=(NBT,),
        out_shape=(
            jax.ShapeDtypeStruct((2, B, H), f32),
            jax.ShapeDtypeStruct((2, 2, H), f32),
        ),
        in_specs=[
            pl.BlockSpec((TB, NT), lambda bi: (bi, 0)),
            pl.BlockSpec((2, K1_IN, H), lambda bi: (0, 0, 0)),
            pl.BlockSpec((2, 1, H), lambda bi: (0, 0, 0)),
        ],
        out_specs=(
            pl.BlockSpec((2, TB, H), lambda bi: (0, bi, 0)),
            pl.BlockSpec((2, 2, H), lambda bi: (0, 0, 0)),
        ),
    )(x, p, b0)

    def mid_layer(raw, st, g1, be1, g2, be2, w1, w2, b1, b2):
        g = jnp.stack([g1, g2]).reshape(2, 1, H)
        be = jnp.stack([be1, be2]).reshape(2, 1, H)
        w = jnp.stack([w1, w2])
        b = jnp.stack([b1, b2]).reshape(2, 1, H)
        return pl.pallas_call(
            _mid_kernel,
            grid=(2, NBT),
            out_shape=(
                jax.ShapeDtypeStruct((2, B, H), f32),
                jax.ShapeDtypeStruct((2, 2, H), f32),
            ),
            in_specs=[
                pl.BlockSpec((1, TB, H), lambda t, bi: (t, bi, 0)),
                pl.BlockSpec((1, 2, H), lambda t, bi: (t, 0, 0)),
                pl.BlockSpec((1, 1, H), lambda t, bi: (t, 0, 0)),
                pl.BlockSpec((1, 1, H), lambda t, bi: (t, 0, 0)),
                pl.BlockSpec((1, H, H), lambda t, bi: (t, 0, 0)),
                pl.BlockSpec((1, 1, H), lambda t, bi: (t, 0, 0)),
            ],
            out_specs=(
                pl.BlockSpec((1, TB, H), lambda t, bi: (t, bi, 0)),
                pl.BlockSpec((1, 2, H), lambda t, bi: (t, 0, 0)),
            ),
        )(raw, st, g, be, w, b)

    raw2, st2 = mid_layer(raw1, st1, fc1_g0, fc1_be0, fc2_g0, fc2_be0,
                          fc1_W1, fc2_W1, fc1_b1, fc2_b1)
    raw3, st3 = mid_layer(raw2, st2, fc1_g1, fc1_be1, fc2_g1, fc2_be1,
                          fc1_W2, fc2_W2, fc1_b2, fc2_b2)

    g3 = jnp.stack([fc1_g2, fc2_g2]).reshape(2, 1, H)
    be3 = jnp.stack([fc1_be2, fc2_be2]).reshape(2, 1, H)
    w3 = jnp.stack([fc1_W3, fc2_W3])
    b3 = jnp.stack([fc1_b3, fc2_b3]).reshape(2, 1, 1)

    out = pl.pallas_call(
        _last_kernel,
        grid=(2, NBT),
        out_shape=jax.ShapeDtypeStruct((2, B, 1), f32),
        in_specs=[
            pl.BlockSpec((1, TB, H), lambda t, bi: (t, bi, 0)),
            pl.BlockSpec((1, 2, H), lambda t, bi: (t, 0, 0)),
            pl.BlockSpec((1, 1, H), lambda t, bi: (t, 0, 0)),
            pl.BlockSpec((1, 1, H), lambda t, bi: (t, 0, 0)),
            pl.BlockSpec((1, H, 1), lambda t, bi: (t, 0, 0)),
            pl.BlockSpec((1, 1, 1), lambda t, bi: (t, 0, 0)),
        ],
        out_specs=pl.BlockSpec((1, TB, 1), lambda t, bi: (t, bi, 0)),
    )(raw3, st3, g3, be3, w3, b3)

    return (out[0], out[1])
